# baseline (speedup 1.0000x reference)
import functools

import jax
import jax.numpy as jnp
from jax import lax
from jax.experimental import pallas as pl
from jax.experimental.pallas import tpu as pltpu
from jax.experimental.pallas import tpu_sc as plsc

NUM_CORES = 2
NUM_SUBCORES = 16
NW = NUM_CORES * NUM_SUBCORES


def _tc_part(table, nb, t):
    d = table.shape[1]
    bt = 512
    grid = (t // bt,)

    def body(tab_ref, out_ref):
        out_ref[...] = jnp.broadcast_to(tab_ref[...][None], (nb, bt, d))

    return pl.pallas_call(
        body,
        grid=grid,
        in_specs=[pl.BlockSpec((bt, d), lambda i: (i, 0))],
        out_specs=pl.BlockSpec((nb, bt, d), lambda i: (0, i, 0)),
        out_shape=jax.ShapeDtypeStruct((nb, t, d), jnp.float32),
    )(table)


def _sc_part(table, t):
    d = table.shape[1]
    rows_per_w = t // NW
    chunk = min(rows_per_w, 64)
    n_chunks = rows_per_w // chunk

    mesh = plsc.VectorSubcoreMesh(core_axis_name="c", subcore_axis_name="s")

    @functools.partial(
        pl.kernel,
        mesh=mesh,
        out_type=jax.ShapeDtypeStruct((t, d), jnp.float32),
        scratch_types=[pltpu.VMEM((chunk, d), jnp.float32)],
    )
    def body(table_hbm, out_hbm, buf):
        wid = lax.axis_index("s") * NUM_CORES + lax.axis_index("c")
        base = wid * rows_per_w
        for c in range(n_chunks):
            r0 = base + c * chunk
            pltpu.sync_copy(table_hbm.at[pl.ds(r0, chunk)], buf)
            pltpu.sync_copy(buf, out_hbm.at[pl.ds(r0, chunk)])

    return body(table)


@functools.partial(jax.jit, static_argnums=(1, 2))
def _posemb(table, b, t):
    tc = _tc_part(table, b - 1, t)
    sc = _sc_part(table, t)
    return jnp.concatenate([tc, sc[None]], axis=0)


def kernel(x, positional_emb):
    b, t = x.shape
    assert t % NW == 0
    return _posemb(positional_emb, b, t)


# speedup vs baseline: 1.3601x; 1.3601x over previous
import functools

import jax
import jax.numpy as jnp
from jax import lax
from jax.experimental import pallas as pl
from jax.experimental.pallas import tpu as pltpu
from jax.experimental.pallas import tpu_sc as plsc

NUM_CORES = 2
NUM_SUBCORES = 16


@functools.partial(jax.jit, static_argnums=(1, 2))
def _posemb_sc(table, b, t):
    d = table.shape[1]
    rows_per_core = t // NUM_CORES
    chunk = 256
    n_chunks = rows_per_core // chunk

    mesh = plsc.VectorSubcoreMesh(core_axis_name="c", subcore_axis_name="s")

    @functools.partial(
        pl.kernel,
        mesh=mesh,
        out_type=jax.ShapeDtypeStruct((b * t, d), jnp.float32),
        scratch_types=[
            pltpu.VMEM_SHARED((chunk, d), jnp.float32),
            pltpu.VMEM_SHARED((chunk, d), jnp.float32),
            pltpu.SemaphoreType.DMA,
            pltpu.SemaphoreType.DMA,
            pltpu.SemaphoreType.DMA,
            pltpu.SemaphoreType.DMA,
        ],
    )
    def body(table_hbm, out_hbm, buf0, buf1, rs0, rs1, ws0, ws1):
        cid = lax.axis_index("c")
        sid = lax.axis_index("s")
        bufs = (buf0, buf1)
        rsems = (rs0, rs1)
        wsems = (ws0, ws1)

        @pl.when(sid == 0)
        def _():
            base = cid * rows_per_core

            def start_read(c):
                r0 = base + c * chunk
                return pltpu.async_copy(
                    table_hbm.at[pl.ds(r0, chunk)], bufs[c % 2], rsems[c % 2])

            def start_writes(c):
                r0 = base + c * chunk
                return [
                    pltpu.async_copy(
                        bufs[c % 2], out_hbm.at[pl.ds(bi * t + r0, chunk)],
                        wsems[c % 2])
                    for bi in range(b)
                ]

            rd = start_read(0)
            for c in range(n_chunks):
                rd.wait()
                if c + 1 < n_chunks:
                    rd = start_read(c + 1)
                for w in start_writes(c):
                    w.wait()

    return body(table)


def kernel(x, positional_emb):
    b, t = x.shape
    out = _posemb_sc(positional_emb, b, t)
    return out.reshape(b, t, positional_emb.shape[1])


# speedup vs baseline: 1.9762x; 1.4530x over previous
import functools

import jax
import jax.numpy as jnp
from jax import lax
from jax.experimental import pallas as pl
from jax.experimental.pallas import tpu as pltpu
from jax.experimental.pallas import tpu_sc as plsc

NUM_CORES = 2
NUM_SUBCORES = 16
NW = NUM_CORES * NUM_SUBCORES

T_STREAM = 2560
SP_CHUNK = 256


@functools.partial(jax.jit, static_argnums=(1, 2))
def _posemb_sc(table, b, t):
    d = table.shape[1]
    srows = T_STREAM // NW
    sp_total = t - T_STREAM
    sp_per_core = sp_total // NUM_CORES
    n_sp_chunks = sp_per_core // SP_CHUNK

    mesh = plsc.VectorSubcoreMesh(core_axis_name="c", subcore_axis_name="s")

    @functools.partial(
        pl.kernel,
        mesh=mesh,
        out_type=jax.ShapeDtypeStruct((b * t, d), jnp.float32),
        scratch_types=[
            pltpu.VMEM((srows, d), jnp.float32),
            pltpu.VMEM_SHARED((SP_CHUNK, d), jnp.float32),
            pltpu.VMEM_SHARED((SP_CHUNK, d), jnp.float32),
            pltpu.SemaphoreType.DMA,
            pltpu.SemaphoreType.DMA,
            pltpu.SemaphoreType.DMA,
            pltpu.SemaphoreType.DMA,
            pltpu.SemaphoreType.DMA,
            pltpu.SemaphoreType.DMA,
        ],
    )
    def body(table_hbm, out_hbm, sbuf, pbuf0, pbuf1, srs, sws, rs0, rs1,
             ws0, ws1):
        cid = lax.axis_index("c")
        sid = lax.axis_index("s")
        wid = sid * NUM_CORES + cid
        pbufs = (pbuf0, pbuf1)
        rsems = (rs0, rs1)
        wsems = (ws0, ws1)

        sp_base = T_STREAM + cid * sp_per_core

        def sp_read(c):
            r0 = sp_base + c * SP_CHUNK
            return pltpu.async_copy(
                table_hbm.at[pl.ds(r0, SP_CHUNK)], pbufs[c % 2], rsems[c % 2])

        def sp_writes(c):
            r0 = sp_base + c * SP_CHUNK
            return [
                pltpu.async_copy(
                    pbufs[c % 2], out_hbm.at[pl.ds(bi * t + r0, SP_CHUNK)],
                    wsems[c % 2])
                for bi in range(b)
            ]

        base = wid * srows
        srd = pltpu.async_copy(table_hbm.at[pl.ds(base, srows)], sbuf, srs)

        is_captain = sid == NUM_SUBCORES - 1

        @pl.when(is_captain)
        def _():
            rd0 = sp_read(0)
            rd1 = None if n_sp_chunks < 2 else sp_read(1)
            rd0.wait()
            del rd0

        srd.wait()
        swr = [
            pltpu.async_copy(sbuf, out_hbm.at[pl.ds(bi * t + base, srows)],
                             sws)
            for bi in range(b)
        ]

        @pl.when(is_captain)
        def _():
            for c in range(n_sp_chunks):
                if c >= 1:
                    pltpu.make_async_copy(
                        table_hbm.at[pl.ds(sp_base + c * SP_CHUNK, SP_CHUNK)],
                        pbufs[c % 2], rsems[c % 2]).wait()
                wr = sp_writes(c)
                if c + 2 < n_sp_chunks:
                    pass
                for w in wr:
                    w.wait()
                if c + 2 < n_sp_chunks:
                    sp_read(c + 2)

        for w in swr:
            w.wait()

    return body(table)


def kernel(x, positional_emb):
    b, t = x.shape
    out = _posemb_sc(positional_emb, b, t)
    return out.reshape(b, t, positional_emb.shape[1])
